# Initial kernel scaffold; baseline (speedup 1.0000x reference)
import jax
import jax.numpy as jnp
from jax.experimental import pallas as pl


def kernel(input, weight, bias):
    raise NotImplementedError("write your pallas kernel here")


# speedup vs baseline: 1.0876x; 1.0876x over previous
import jax
import jax.numpy as jnp
from jax.experimental import pallas as pl
from jax.experimental.pallas import tpu as pltpu

BM = 512
BN = 1024


def _linear_kernel(x_ref, w_ref, b_ref, o_ref):
    acc = jax.lax.dot_general(
        x_ref[...], w_ref[...],
        dimension_numbers=(((1,), (1,)), ((), ())),
        preferred_element_type=jnp.float32,
    )
    o_ref[...] = acc + b_ref[...]


def kernel(input, weight, bias):
    M, K = input.shape
    N = weight.shape[0]
    grid = (N // BN, M // BM)
    return pl.pallas_call(
        _linear_kernel,
        grid=grid,
        in_specs=[
            pl.BlockSpec((BM, K), lambda j, i: (i, 0)),
            pl.BlockSpec((BN, K), lambda j, i: (j, 0)),
            pl.BlockSpec((1, BN), lambda j, i: (0, j)),
        ],
        out_specs=pl.BlockSpec((BM, BN), lambda j, i: (i, j)),
        out_shape=jax.ShapeDtypeStruct((M, N), jnp.float32),
        compiler_params=pltpu.CompilerParams(
            dimension_semantics=("parallel", "arbitrary"),
            vmem_limit_bytes=56 * 1024 * 1024,
        ),
        name="safe_linear",
    )(input, weight, bias.reshape(1, N))


# speedup vs baseline: 1.1055x; 1.0165x over previous
import jax
import jax.numpy as jnp
from jax.experimental import pallas as pl
from jax.experimental.pallas import tpu as pltpu

BM = 512
BN = 2048


def _linear_kernel(x_ref, w_hbm, b_ref, o_ref, w_vmem, w_sem):
    j = pl.program_id(0)

    @pl.when(pl.program_id(1) == 0)
    def _load_w():
        cp = pltpu.make_async_copy(
            w_hbm.at[pl.ds(j * BN, BN), :], w_vmem, w_sem)
        cp.start()
        cp.wait()

    acc = jax.lax.dot_general(
        x_ref[...], w_vmem[...],
        dimension_numbers=(((1,), (1,)), ((), ())),
        preferred_element_type=jnp.float32,
    )
    o_ref[...] = acc + b_ref[...]


def kernel(input, weight, bias):
    M, K = input.shape
    N = weight.shape[0]
    grid = (N // BN, M // BM)
    return pl.pallas_call(
        _linear_kernel,
        grid=grid,
        in_specs=[
            pl.BlockSpec((BM, K), lambda j, i: (i, 0)),
            pl.BlockSpec(memory_space=pl.ANY),
            pl.BlockSpec((1, BN), lambda j, i: (0, j)),
        ],
        out_specs=pl.BlockSpec((BM, BN), lambda j, i: (i, j)),
        out_shape=jax.ShapeDtypeStruct((M, N), jnp.float32),
        scratch_shapes=[
            pltpu.VMEM((BN, 4096), jnp.float32),
            pltpu.SemaphoreType.DMA,
        ],
        compiler_params=pltpu.CompilerParams(
            dimension_semantics=("arbitrary", "arbitrary"),
            vmem_limit_bytes=60 * 1024 * 1024,
        ),
        name="safe_linear",
    )(input, weight, bias.reshape(1, N))
